# baseline (speedup 1.0000x reference)
import jax
import jax.numpy as jnp
from jax.experimental import pallas as pl
from jax.experimental.pallas import tpu as pltpu

_DIM = 4096
_WIDTH = 4
_BATCH = 128
_SEQ = 4
_M = 2048
_RB = 8
_NBLK = _M // _RB


def _body(idx_ref, cs_copy_ref, cs_gat_ref, x_ref, w_ref, b_ref,
          state_out_ref, out_ref, winner_ref):
    i = pl.program_id(0)

    @pl.when(i == 0)
    def _():
        def scat(b, carry):
            winner_ref[idx_ref[b]] = b
            return carry
        jax.lax.fori_loop(0, _BATCH, scat, 0)

    state_out_ref[...] = cs_copy_ref[...]

    for r in range(_RB):
        m = i * _RB + r
        wc = jnp.clip(winner_ref[m], 0, _BATCH - 1)

        @pl.when(idx_ref[wc] == m)
        def _():
            state_out_ref[r, :, :] = x_ref[wc, 1:, :]

    @pl.when(i < _BATCH)
    def _():
        bb = jnp.minimum(i, _BATCH - 1)
        st = cs_gat_ref[0]
        xb = x_ref[bb]
        xn = jnp.concatenate([st, xb], axis=0)
        acc = jnp.broadcast_to(b_ref[0][None, :], (_SEQ, _DIM))
        for k in range(_WIDTH):
            acc = acc + xn[k:k + _SEQ, :] * w_ref[k][None, :]
        out_ref[0] = acc * jax.nn.sigmoid(acc)


def kernel(x, conv_state, conv_state_indices, weight, bias):
    bias2d = bias.reshape(1, _DIM)
    grid_spec = pltpu.PrefetchScalarGridSpec(
        num_scalar_prefetch=1,
        grid=(_NBLK,),
        in_specs=[
            pl.BlockSpec((_RB, _WIDTH - 1, _DIM), lambda i, idx: (i, 0, 0)),
            pl.BlockSpec(
                (1, _WIDTH - 1, _DIM),
                lambda i, idx: (idx[jnp.minimum(i, _BATCH - 1)], 0, 0)),
            pl.BlockSpec((_BATCH, _SEQ, _DIM), lambda i, idx: (0, 0, 0)),
            pl.BlockSpec((_WIDTH, _DIM), lambda i, idx: (0, 0)),
            pl.BlockSpec((1, _DIM), lambda i, idx: (0, 0)),
        ],
        out_specs=[
            pl.BlockSpec((_RB, _WIDTH - 1, _DIM), lambda i, idx: (i, 0, 0)),
            pl.BlockSpec(
                (1, _SEQ, _DIM),
                lambda i, idx: (jnp.minimum(i, _BATCH - 1), 0, 0)),
        ],
        scratch_shapes=[pltpu.SMEM((_M,), jnp.int32)],
    )
    state_out, out = pl.pallas_call(
        _body,
        grid_spec=grid_spec,
        out_shape=[
            jax.ShapeDtypeStruct((_M, _WIDTH - 1, _DIM), jnp.float32),
            jax.ShapeDtypeStruct((_BATCH, _SEQ, _DIM), jnp.float32),
        ],
        compiler_params=pltpu.CompilerParams(
            dimension_semantics=("arbitrary",),
        ),
    )(conv_state_indices, conv_state, conv_state, x, weight, bias2d)
    return out, state_out
